# Initial kernel scaffold; baseline (speedup 1.0000x reference)
import functools

import jax
import jax.numpy as jnp
from jax import lax
from jax.experimental import pallas as pl
from jax.experimental.pallas import tpu as pltpu
from jax.experimental.pallas import tpu_sc as plsc

N = 50000
E = 800000
HID = 10
MSG = 10
NCLS = 16
NT = 8

LP = 16
NY = 50176
BN = 512
NM = 51200
ROWS_PER_TILE = NM // 16
ZCOPIES = ROWS_PER_TILE // 128
NC, NS = 2, 16
NW = NC * NS
CH = 128
NCHUNK = 196
EPW = NCHUNK * CH
EPAD = NW * EPW


def _ytab_body(f_ref, at_ref, y_ref):
    f = f_ref[...]
    for t in range(NT):
        y_ref[t] = lax.dot_general(
            f, at_ref[t], (((1,), (1,)), ((), ())),
            preferred_element_type=jnp.float32)


def _compute_ytab(f16, at_pad):
    return pl.pallas_call(
        _ytab_body,
        grid=(NY // BN,),
        in_specs=[
            pl.BlockSpec((BN, LP), lambda i: (i, 0)),
            pl.BlockSpec((NT, LP, LP), lambda i: (0, 0, 0)),
        ],
        out_specs=pl.BlockSpec((NT, BN, LP), lambda i: (0, i, 0)),
        out_shape=jax.ShapeDtypeStruct((NT, NY, LP), jnp.float32),
    )(f16, at_pad)


def _edge_body(y_hbm, src_hbm, typ_hbm, dst_hbm, out_hbm,
               src_v, typ_v, dst_v, rows_v, m_sh, sem):
    c = lax.axis_index("c")
    s = lax.axis_index("s")
    wid = s * NC + c

    def zrow(j, _):
        rows_v[j, :] = jnp.zeros((LP,), jnp.float32)
        return 0
    lax.fori_loop(0, CH, zrow, 0, unroll=8)

    def zcopy(k, _):
        pltpu.sync_copy(rows_v, m_sh.at[pl.ds(s * ROWS_PER_TILE + k * CH, CH)])
        return 0
    lax.fori_loop(0, ZCOPIES, zcopy, 0)

    base = wid * NCHUNK
    pltpu.sync_copy(src_hbm.at[pl.ds(base, NCHUNK)], src_v)
    pltpu.sync_copy(typ_hbm.at[pl.ds(base, NCHUNK)], typ_v)
    pltpu.sync_copy(dst_hbm.at[pl.ds(base, NCHUNK)], dst_v)

    def gidx(ch, _):
        for j in range(CH // 16):
            sl = pl.ds(j * 16, 16)
            src_v[ch, sl] = typ_v[ch, sl] * NY + src_v[ch, sl]
        return 0
    lax.fori_loop(0, NCHUNK, gidx, 0)

    plsc.subcore_barrier()

    def step(ch, _):
        pltpu.async_copy(y_hbm.at[src_v.at[ch]], rows_v, sem).wait()
        pltpu.sync_copy(rows_v, m_sh.at[dst_v.at[ch]], add=True)
        return 0
    lax.fori_loop(0, NCHUNK, step, 0)

    plsc.subcore_barrier()

    rs = pl.ds(s * ROWS_PER_TILE, ROWS_PER_TILE)
    pltpu.sync_copy(m_sh.at[rs], out_hbm.at[c, rs])


def _edge_aggregate(y2, src2, typ2, dst2):
    mesh = plsc.VectorSubcoreMesh(core_axis_name="c", subcore_axis_name="s")
    run = pl.kernel(
        _edge_body,
        out_type=jax.ShapeDtypeStruct((NC, NM, LP), jnp.float32),
        mesh=mesh,
        scratch_types=[
            pltpu.VMEM((NCHUNK, CH), jnp.int32),
            pltpu.VMEM((NCHUNK, CH), jnp.int32),
            pltpu.VMEM((NCHUNK, CH), jnp.int32),
            pltpu.VMEM((CH, LP), jnp.float32),
            pltpu.VMEM_SHARED((NM, LP), jnp.float32),
            pltpu.SemaphoreType.DMA,
        ],
    )
    return run(y2, src2, typ2, dst2)


def _gru_body(m_ref, f_ref, w_ref, b_ref, o_ref):
    m = m_ref[0] + m_ref[1]
    f = f_ref[...]
    wir, wiz, win = w_ref[0], w_ref[1], w_ref[2]
    whr, whz, whn = w_ref[3], w_ref[4], w_ref[5]
    wout = w_ref[6]
    dn = (((1,), (0,)), ((), ()))
    dot = functools.partial(lax.dot_general, dimension_numbers=dn,
                            preferred_element_type=jnp.float32)
    br, bz = b_ref[0], b_ref[1]
    bin_, bhn = b_ref[2], b_ref[3]
    bout = b_ref[4]
    r = jax.nn.sigmoid(dot(m, wir) + dot(f, whr) + br)
    z = jax.nn.sigmoid(dot(m, wiz) + dot(f, whz) + bz)
    n = jnp.tanh(dot(m, win) + bin_ + r * (dot(f, whn) + bhn))
    h = (1.0 - z) * n + z * f
    o_ref[...] = dot(h, wout) + bout


def _gru_readout(m_part, f16, w_stack, b_stack):
    return pl.pallas_call(
        _gru_body,
        grid=(NY // BN,),
        in_specs=[
            pl.BlockSpec((NC, BN, LP), lambda i: (0, i, 0)),
            pl.BlockSpec((BN, LP), lambda i: (i, 0)),
            pl.BlockSpec((7, LP, LP), lambda i: (0, 0, 0)),
            pl.BlockSpec((5, 1, LP), lambda i: (0, 0, 0)),
        ],
        out_specs=pl.BlockSpec((BN, LP), lambda i: (i, 0)),
        out_shape=jax.ShapeDtypeStruct((NY, LP), jnp.float32),
    )(m_part, f16, w_stack, b_stack)


def kernel(features, edge_index, edge_types, edge_table,
           W_ih, W_hh, b_ih, b_hh, W_out, b_out):
    f32 = jnp.float32
    f16 = jnp.zeros((NY, LP), f32).at[:N, :HID].set(features)
    at_pad = (jnp.zeros((NT, LP, LP), f32)
              .at[:, :MSG, :HID].set(edge_table.reshape(NT, MSG, HID)))

    src = edge_index[0]
    dst = edge_index[1]
    pad = EPAD - E
    src2 = jnp.concatenate([src, jnp.zeros((pad,), jnp.int32)]).reshape(-1, CH)
    typ2 = jnp.concatenate([edge_types, jnp.zeros((pad,), jnp.int32)]).reshape(-1, CH)
    dst2 = jnp.concatenate([dst, jnp.full((pad,), N, jnp.int32)]).reshape(-1, CH)

    def wpad(w):
        return jnp.zeros((LP, LP), f32).at[:w.shape[1], :w.shape[0]].set(w.T)
    w_stack = jnp.stack([
        wpad(W_ih[0:HID]), wpad(W_ih[HID:2 * HID]), wpad(W_ih[2 * HID:]),
        wpad(W_hh[0:HID]), wpad(W_hh[HID:2 * HID]), wpad(W_hh[2 * HID:]),
        jnp.zeros((LP, LP), f32).at[:HID, :NCLS].set(W_out.T),
    ])

    def bpad(b):
        return jnp.zeros((1, LP), f32).at[0, :b.shape[0]].set(b)
    b_stack = jnp.stack([
        bpad(b_ih[0:HID] + b_hh[0:HID]),
        bpad(b_ih[HID:2 * HID] + b_hh[HID:2 * HID]),
        bpad(b_ih[2 * HID:]),
        bpad(b_hh[2 * HID:]),
        bpad(b_out),
    ])

    ytab = _compute_ytab(f16, at_pad)
    y2 = ytab.reshape(NT * NY, LP)

    m_part = _edge_aggregate(y2, src2, typ2, dst2)

    out = _gru_readout(m_part[:, :NY], f16, w_stack, b_stack)
    return out[:N, :NCLS]


# speedup vs baseline: 12.2822x; 12.2822x over previous
import functools

import jax
import jax.numpy as jnp
from jax import lax
from jax.experimental import pallas as pl
from jax.experimental.pallas import tpu as pltpu
from jax.experimental.pallas import tpu_sc as plsc

N = 50000
E = 800000
HID = 10
MSG = 10
NCLS = 16
NT = 8

LP = 16
NY = 50176
BN = 512
NM = 51200
ROWS_PER_TILE = NM // 16
ZCOPIES = ROWS_PER_TILE // 128
NC, NS = 2, 16
NW = NC * NS
CH = 128
NCHUNK = 196
EPW = NCHUNK * CH
EPAD = NW * EPW


def _ytab_body(f_ref, at_ref, y_ref):
    f = f_ref[...]
    for t in range(NT):
        y_ref[t] = lax.dot_general(
            f, at_ref[t], (((1,), (1,)), ((), ())),
            preferred_element_type=jnp.float32)


def _compute_ytab(f16, at_pad):
    return pl.pallas_call(
        _ytab_body,
        grid=(NY // BN,),
        in_specs=[
            pl.BlockSpec((BN, LP), lambda i: (i, 0)),
            pl.BlockSpec((NT, LP, LP), lambda i: (0, 0, 0)),
        ],
        out_specs=pl.BlockSpec((NT, BN, LP), lambda i: (0, i, 0)),
        out_shape=jax.ShapeDtypeStruct((NT, NY, LP), jnp.float32),
    )(f16, at_pad)


def _edge_body(y_hbm, src_hbm, typ_hbm, dst_hbm, out_hbm,
               src_v, typ_v, dst_v, rows_v, m_sh, sem):
    c = lax.axis_index("c")
    s = lax.axis_index("s")
    wid = s * NC + c

    def zrow(j, _):
        rows_v[j, :] = jnp.zeros((LP,), jnp.float32)
        return 0
    lax.fori_loop(0, CH, zrow, 0, unroll=8)

    def zcopy(k, _):
        pltpu.sync_copy(rows_v, m_sh.at[pl.ds(s * ROWS_PER_TILE + k * CH, CH)])
        return 0
    lax.fori_loop(0, ZCOPIES, zcopy, 0)

    pltpu.sync_copy(src_hbm.at[wid], src_v)
    pltpu.sync_copy(typ_hbm.at[wid], typ_v)
    pltpu.sync_copy(dst_hbm.at[wid], dst_v)

    def gidx(ch, _):
        for j in range(CH // 16):
            sl = pl.ds(j * 16, 16)
            src_v[ch, sl] = typ_v[ch, sl] * NY + src_v[ch, sl]
        return 0
    lax.fori_loop(0, NCHUNK, gidx, 0)

    plsc.subcore_barrier()

    def step(ch, _):
        pltpu.async_copy(y_hbm.at[src_v.at[ch]], rows_v, sem).wait()
        pltpu.sync_copy(rows_v, m_sh.at[dst_v.at[ch]], add=True)
        return 0
    lax.fori_loop(0, NCHUNK, step, 0)

    plsc.subcore_barrier()

    rs = pl.ds(s * ROWS_PER_TILE, ROWS_PER_TILE)
    pltpu.sync_copy(m_sh.at[rs], out_hbm.at[c, rs])


def _edge_aggregate(y2, src2, typ2, dst2):
    mesh = plsc.VectorSubcoreMesh(core_axis_name="c", subcore_axis_name="s")
    run = pl.kernel(
        _edge_body,
        out_type=jax.ShapeDtypeStruct((NC, NM, LP), jnp.float32),
        mesh=mesh,
        scratch_types=[
            pltpu.VMEM((NCHUNK, CH), jnp.int32),
            pltpu.VMEM((NCHUNK, CH), jnp.int32),
            pltpu.VMEM((NCHUNK, CH), jnp.int32),
            pltpu.VMEM((CH, LP), jnp.float32),
            pltpu.VMEM_SHARED((NM, LP), jnp.float32),
            pltpu.SemaphoreType.DMA,
        ],
        compiler_params=pltpu.CompilerParams(use_tc_tiling_on_sc=False),
    )
    return run(y2, src2, typ2, dst2)


def _gru_body(m_ref, f_ref, w_ref, b_ref, o_ref):
    m = m_ref[0] + m_ref[1]
    f = f_ref[...]
    wir, wiz, win = w_ref[0], w_ref[1], w_ref[2]
    whr, whz, whn = w_ref[3], w_ref[4], w_ref[5]
    wout = w_ref[6]
    dn = (((1,), (0,)), ((), ()))
    dot = functools.partial(lax.dot_general, dimension_numbers=dn,
                            preferred_element_type=jnp.float32)
    br, bz = b_ref[0], b_ref[1]
    bin_, bhn = b_ref[2], b_ref[3]
    bout = b_ref[4]
    r = jax.nn.sigmoid(dot(m, wir) + dot(f, whr) + br)
    z = jax.nn.sigmoid(dot(m, wiz) + dot(f, whz) + bz)
    n = jnp.tanh(dot(m, win) + bin_ + r * (dot(f, whn) + bhn))
    h = (1.0 - z) * n + z * f
    o_ref[...] = dot(h, wout) + bout


def _gru_readout(m_part, f16, w_stack, b_stack):
    return pl.pallas_call(
        _gru_body,
        grid=(NY // BN,),
        in_specs=[
            pl.BlockSpec((NC, BN, LP), lambda i: (0, i, 0)),
            pl.BlockSpec((BN, LP), lambda i: (i, 0)),
            pl.BlockSpec((7, LP, LP), lambda i: (0, 0, 0)),
            pl.BlockSpec((5, 1, LP), lambda i: (0, 0, 0)),
        ],
        out_specs=pl.BlockSpec((BN, LP), lambda i: (i, 0)),
        out_shape=jax.ShapeDtypeStruct((NY, LP), jnp.float32),
    )(m_part, f16, w_stack, b_stack)


def kernel(features, edge_index, edge_types, edge_table,
           W_ih, W_hh, b_ih, b_hh, W_out, b_out):
    f32 = jnp.float32
    f16 = jnp.zeros((NY, LP), f32).at[:N, :HID].set(features)
    at_pad = (jnp.zeros((NT, LP, LP), f32)
              .at[:, :MSG, :HID].set(edge_table.reshape(NT, MSG, HID)))

    src = edge_index[0]
    dst = edge_index[1]
    pad = EPAD - E
    src2 = jnp.concatenate(
        [src, jnp.zeros((pad,), jnp.int32)]).reshape(NW, NCHUNK, CH)
    typ2 = jnp.concatenate(
        [edge_types, jnp.zeros((pad,), jnp.int32)]).reshape(NW, NCHUNK, CH)
    dst2 = jnp.concatenate(
        [dst, jnp.full((pad,), N, jnp.int32)]).reshape(NW, NCHUNK, CH)

    def wpad(w):
        return jnp.zeros((LP, LP), f32).at[:w.shape[1], :w.shape[0]].set(w.T)
    w_stack = jnp.stack([
        wpad(W_ih[0:HID]), wpad(W_ih[HID:2 * HID]), wpad(W_ih[2 * HID:]),
        wpad(W_hh[0:HID]), wpad(W_hh[HID:2 * HID]), wpad(W_hh[2 * HID:]),
        jnp.zeros((LP, LP), f32).at[:HID, :NCLS].set(W_out.T),
    ])

    def bpad(b):
        return jnp.zeros((1, LP), f32).at[0, :b.shape[0]].set(b)
    b_stack = jnp.stack([
        bpad(b_ih[0:HID] + b_hh[0:HID]),
        bpad(b_ih[HID:2 * HID] + b_hh[HID:2 * HID]),
        bpad(b_ih[2 * HID:]),
        bpad(b_hh[2 * HID:]),
        bpad(b_out),
    ])

    ytab = _compute_ytab(f16, at_pad)
    y2 = ytab.reshape(NT * NY, LP)

    m_part = _edge_aggregate(y2, src2, typ2, dst2)

    out = _gru_readout(m_part[:, :NY], f16, w_stack, b_stack)
    return out[:N, :NCLS]


# speedup vs baseline: 14.7816x; 1.2035x over previous
import functools

import jax
import jax.numpy as jnp
from jax import lax
from jax.experimental import pallas as pl
from jax.experimental.pallas import tpu as pltpu
from jax.experimental.pallas import tpu_sc as plsc

N = 50000
E = 800000
HID = 10
MSG = 10
NCLS = 16
NT = 8

LP = 16
NY = 50176
BN = 512
NM = 51200
ROWS_PER_TILE = NM // 16
ZCOPIES = ROWS_PER_TILE // 128
NC, NS = 2, 16
NW = NC * NS
CH = 128
NCHUNK = 196
EPW = NCHUNK * CH
EPAD = NW * EPW


def _ytab_body(f_ref, at_ref, y_ref):
    f = f_ref[...]
    for t in range(NT):
        y_ref[t] = lax.dot_general(
            f, at_ref[t], (((1,), (1,)), ((), ())),
            preferred_element_type=jnp.float32)


def _compute_ytab(f16, at_pad):
    return pl.pallas_call(
        _ytab_body,
        grid=(NY // BN,),
        in_specs=[
            pl.BlockSpec((BN, LP), lambda i: (i, 0)),
            pl.BlockSpec((NT, LP, LP), lambda i: (0, 0, 0)),
        ],
        out_specs=pl.BlockSpec((NT, BN, LP), lambda i: (0, i, 0)),
        out_shape=jax.ShapeDtypeStruct((NT, NY, LP), jnp.float32),
    )(f16, at_pad)


def _gidx_body(src_ref, typ_ref, g_ref):
    g_ref[...] = typ_ref[...] * NY + src_ref[...]


def _compute_gidx(src2, typ2):
    rows = EPAD // CH
    blk = rows // 8
    return pl.pallas_call(
        _gidx_body,
        grid=(8,),
        in_specs=[pl.BlockSpec((blk, CH), lambda i: (i, 0)),
                  pl.BlockSpec((blk, CH), lambda i: (i, 0))],
        out_specs=pl.BlockSpec((blk, CH), lambda i: (i, 0)),
        out_shape=jax.ShapeDtypeStruct((rows, CH), jnp.int32),
    )(src2.reshape(rows, CH), typ2.reshape(rows, CH)).reshape(NW, EPW)


G = 14
NGROUPS = NCHUNK // G
GR = G * CH


def _edge_body(y_hbm, g_hbm, dst_hbm, out_hbm,
               gi_v, di_v, rows_v, m_sh, sem_i, sem_g, sem_s):
    c = lax.axis_index("c")
    s = lax.axis_index("s")
    wid = s * NC + c

    def zrow(j, _):
        rows_v[0, j, :] = jnp.zeros((LP,), jnp.float32)
        return 0
    lax.fori_loop(0, CH, zrow, 0, unroll=8)

    zsrc = rows_v.at[0, pl.ds(0, CH)]

    def zcopy(k, _):
        pltpu.sync_copy(zsrc, m_sh.at[pl.ds(s * ROWS_PER_TILE + k * CH, CH)])
        return 0
    lax.fori_loop(0, ZCOPIES, zcopy, 0)

    def fire_idx(g, b):
        pltpu.async_copy(g_hbm.at[wid, pl.ds(g * GR, GR)], gi_v.at[b], sem_i)
        pltpu.async_copy(dst_hbm.at[wid, pl.ds(g * G, G)], di_v.at[b], sem_i)

    def wait_idx(b):
        pltpu.make_async_copy(g_hbm.at[0, pl.ds(0, GR)], gi_v.at[b],
                              sem_i).wait()
        pltpu.make_async_copy(dst_hbm.at[0, pl.ds(0, G)], di_v.at[b],
                              sem_i).wait()

    def fire_gather(b):
        pltpu.async_copy(y_hbm.at[gi_v.at[b]], rows_v.at[b], sem_g)

    def wait_gather(b):
        pltpu.make_async_copy(y_hbm.at[gi_v.at[b]], rows_v.at[b],
                              sem_g).wait()

    def scatter(b):
        descs = []
        for j in range(G):
            descs.append(pltpu.async_copy(
                rows_v.at[b, pl.ds(j * CH, CH)],
                m_sh.at[di_v.at[b, j]], sem_s, add=True))
        for d in descs:
            d.wait()

    fire_idx(0, 0)

    plsc.subcore_barrier()

    wait_idx(0)
    fire_gather(0)
    fire_idx(1, 1)

    def step(g, _):
        b = lax.rem(g, 2)
        wait_idx(b)
        wait_gather(1 - b)
        fire_gather(b)
        scatter(1 - b)

        @pl.when(g + 1 < NGROUPS)
        def _():
            fire_idx(g + 1, 1 - b)
        return 0
    lax.fori_loop(1, NGROUPS, step, 0)

    lb = (NGROUPS - 1) % 2
    wait_gather(lb)
    scatter(lb)

    plsc.subcore_barrier()

    rs = pl.ds(s * ROWS_PER_TILE, ROWS_PER_TILE)
    pltpu.sync_copy(m_sh.at[rs], out_hbm.at[c, rs])


def _edge_aggregate(y2, g2, dst2):
    mesh = plsc.VectorSubcoreMesh(core_axis_name="c", subcore_axis_name="s")
    run = pl.kernel(
        _edge_body,
        out_type=jax.ShapeDtypeStruct((NC, NM, LP), jnp.float32),
        mesh=mesh,
        scratch_types=[
            pltpu.VMEM((2, GR), jnp.int32),
            pltpu.VMEM((2, G, CH), jnp.int32),
            pltpu.VMEM((2, GR, LP), jnp.float32),
            pltpu.VMEM_SHARED((NM, LP), jnp.float32),
            pltpu.SemaphoreType.DMA,
            pltpu.SemaphoreType.DMA,
            pltpu.SemaphoreType.DMA,
        ],
        compiler_params=pltpu.CompilerParams(use_tc_tiling_on_sc=False),
    )
    return run(y2, g2, dst2)


def _gru_body(m_ref, f_ref, w_ref, b_ref, o_ref):
    m = m_ref[0] + m_ref[1]
    f = f_ref[...]
    wir, wiz, win = w_ref[0], w_ref[1], w_ref[2]
    whr, whz, whn = w_ref[3], w_ref[4], w_ref[5]
    wout = w_ref[6]
    dn = (((1,), (0,)), ((), ()))
    dot = functools.partial(lax.dot_general, dimension_numbers=dn,
                            preferred_element_type=jnp.float32)
    br, bz = b_ref[0], b_ref[1]
    bin_, bhn = b_ref[2], b_ref[3]
    bout = b_ref[4]
    r = jax.nn.sigmoid(dot(m, wir) + dot(f, whr) + br)
    z = jax.nn.sigmoid(dot(m, wiz) + dot(f, whz) + bz)
    n = jnp.tanh(dot(m, win) + bin_ + r * (dot(f, whn) + bhn))
    h = (1.0 - z) * n + z * f
    o_ref[...] = dot(h, wout) + bout


def _gru_readout(m_part, f16, w_stack, b_stack):
    return pl.pallas_call(
        _gru_body,
        grid=(NY // BN,),
        in_specs=[
            pl.BlockSpec((NC, BN, LP), lambda i: (0, i, 0)),
            pl.BlockSpec((BN, LP), lambda i: (i, 0)),
            pl.BlockSpec((7, LP, LP), lambda i: (0, 0, 0)),
            pl.BlockSpec((5, 1, LP), lambda i: (0, 0, 0)),
        ],
        out_specs=pl.BlockSpec((BN, LP), lambda i: (i, 0)),
        out_shape=jax.ShapeDtypeStruct((NY, LP), jnp.float32),
    )(m_part, f16, w_stack, b_stack)


def kernel(features, edge_index, edge_types, edge_table,
           W_ih, W_hh, b_ih, b_hh, W_out, b_out):
    f32 = jnp.float32
    f16 = jnp.zeros((NY, LP), f32).at[:N, :HID].set(features)
    at_pad = (jnp.zeros((NT, LP, LP), f32)
              .at[:, :MSG, :HID].set(edge_table.reshape(NT, MSG, HID)))

    src = edge_index[0]
    dst = edge_index[1]
    pad = EPAD - E
    src2 = jnp.concatenate(
        [src, jnp.zeros((pad,), jnp.int32)]).reshape(NW, NCHUNK, CH)
    typ2 = jnp.concatenate(
        [edge_types, jnp.zeros((pad,), jnp.int32)]).reshape(NW, NCHUNK, CH)
    dst2 = jnp.concatenate(
        [dst, jnp.full((pad,), N, jnp.int32)]).reshape(NW, NCHUNK, CH)

    def wpad(w):
        return jnp.zeros((LP, LP), f32).at[:w.shape[1], :w.shape[0]].set(w.T)
    w_stack = jnp.stack([
        wpad(W_ih[0:HID]), wpad(W_ih[HID:2 * HID]), wpad(W_ih[2 * HID:]),
        wpad(W_hh[0:HID]), wpad(W_hh[HID:2 * HID]), wpad(W_hh[2 * HID:]),
        jnp.zeros((LP, LP), f32).at[:HID, :NCLS].set(W_out.T),
    ])

    def bpad(b):
        return jnp.zeros((1, LP), f32).at[0, :b.shape[0]].set(b)
    b_stack = jnp.stack([
        bpad(b_ih[0:HID] + b_hh[0:HID]),
        bpad(b_ih[HID:2 * HID] + b_hh[HID:2 * HID]),
        bpad(b_ih[2 * HID:]),
        bpad(b_hh[2 * HID:]),
        bpad(b_out),
    ])

    ytab = _compute_ytab(f16, at_pad)
    y2 = ytab.reshape(NT * NY, LP)

    g2 = _compute_gidx(src2, typ2)
    m_part = _edge_aggregate(y2, g2, dst2)

    out = _gru_readout(m_part[:, :NY], f16, w_stack, b_stack)
    return out[:N, :NCLS]
